# baseline (speedup 1.0000x reference)
import functools

import jax
import jax.numpy as jnp
from jax import lax
from jax.experimental import pallas as pl
from jax.experimental.pallas import tpu as pltpu
from jax.experimental.pallas import tpu_sc as plsc

N = 10000
E = 320000
D = 128
B = 1024

NC = 2
NS = 16
L = 16
NW = NC * NS
EPW = E // NW
CHUNK = 80
NCHUNK = EPW // CHUNK
KV = D // L

_mesh = plsc.VectorSubcoreMesh(core_axis_name="c", subcore_axis_name="s")


def _wid():
    return lax.axis_index("s") * NC + lax.axis_index("c")


def _proj_body(xj_ref, xi_ref, wj_ref, wi_ref, bias_ref, lw_ref,
               oj_ref, oi_ref, sj_ref, si_ref):
    oj = jnp.dot(xj_ref[...], wj_ref[...], preferred_element_type=jnp.float32)
    oi = jnp.dot(xi_ref[...], wi_ref[...],
                 preferred_element_type=jnp.float32) + bias_ref[...]
    oj_ref[...] = oj
    oi_ref[...] = oi
    lw_col = lw_ref[...].reshape(D, 1)
    sj_ref[...] = jnp.dot(oj, lw_col, preferred_element_type=jnp.float32)
    si_ref[...] = jnp.dot(oi, lw_col, preferred_element_type=jnp.float32)


def _project(x_j, x_i, w_j, w_i, bias2d, lw2d):
    blk = 2000
    grid = (N // blk,)
    return pl.pallas_call(
        _proj_body,
        grid=grid,
        in_specs=[
            pl.BlockSpec((blk, D), lambda i: (i, 0)),
            pl.BlockSpec((blk, D), lambda i: (i, 0)),
            pl.BlockSpec((D, D), lambda i: (0, 0)),
            pl.BlockSpec((D, D), lambda i: (0, 0)),
            pl.BlockSpec((1, D), lambda i: (0, 0)),
            pl.BlockSpec((1, D), lambda i: (0, 0)),
        ],
        out_specs=[
            pl.BlockSpec((blk, D), lambda i: (i, 0)),
            pl.BlockSpec((blk, D), lambda i: (i, 0)),
            pl.BlockSpec((blk, 1), lambda i: (i, 0)),
            pl.BlockSpec((blk, 1), lambda i: (i, 0)),
        ],
        out_shape=[
            jax.ShapeDtypeStruct((N, D), jnp.float32),
            jax.ShapeDtypeStruct((N, D), jnp.float32),
            jax.ShapeDtypeStruct((N, 1), jnp.float32),
            jax.ShapeDtypeStruct((N, 1), jnp.float32),
        ],
    )(x_j, x_i, w_j, w_i, bias2d, lw2d)


def _alpha_body(xj_hbm, xib_hbm, srcr_hbm, dstr_hbm, lwm_hbm, sj_hbm, si_hbm,
                alpha_hbm, maxp_hbm,
                src_all, dst_all, u_rows, v_rows, a_buf, lwm_v, sj_v, si_v,
                mx_v, lwm_s, sem_u0, sem_v0, sem_u1, sem_v1):
    w = _wid()
    base = w * EPW
    pltpu.sync_copy(lwm_hbm, lwm_v)
    pltpu.sync_copy(sj_hbm, sj_v)
    pltpu.sync_copy(si_hbm, si_v)
    pltpu.sync_copy(srcr_hbm.at[w], src_all)
    pltpu.sync_copy(dstr_hbm.at[w], dst_all)
    for k in range(KV):
        reg = lwm_v[pl.ds(k * L, L)]
        for j in range(L):
            lwm_s[k * L + j] = reg[j]
    lanes = jnp.arange(L, dtype=jnp.int32)
    sems = [(sem_u0, sem_v0), (sem_u1, sem_v1)]

    def fetch(cc, bb):
        su, sv = sems[bb]
        pltpu.async_copy(xj_hbm.at[src_all.at[cc]], u_rows.at[bb], su)
        pltpu.async_copy(xib_hbm.at[dst_all.at[cc]], v_rows.at[bb], sv)

    def drain(cc, bb):
        su, sv = sems[bb]
        pltpu.make_async_copy(
            xj_hbm.at[src_all.at[cc]], u_rows.at[bb], su).wait()
        pltpu.make_async_copy(
            xib_hbm.at[dst_all.at[cc]], v_rows.at[bb], sv).wait()

    fetch(0, 0)

    def chunk_step(c, carry):
        b = lax.rem(c, 2)

        @pl.when(jnp.logical_and(c + 1 < NCHUNK, b == 0))
        def _():
            fetch(c + 1, 1)

        @pl.when(jnp.logical_and(c + 1 < NCHUNK, b == 1))
        def _():
            fetch(c + 1, 0)

        @pl.when(b == 0)
        def _():
            drain(c, 0)

        @pl.when(b == 1)
        def _():
            drain(c, 1)

        def group_step(g, _g):
            e0 = g * L
            sidx = src_all[c, pl.ds(e0, L)]
            didx = dst_all[c, pl.ds(e0, L)]
            vec = (plsc.load_gather(sj_v, [sidx])
                   + plsc.load_gather(si_v, [didx]))
            rowv = e0 + lanes
            ub = u_rows.at[b]
            vb = v_rows.at[b]
            acc = jnp.zeros((L,), jnp.float32)
            for d in range(D):
                dv = jnp.full((L,), d, jnp.int32)
                t = (plsc.load_gather(ub, [rowv, dv])
                     + plsc.load_gather(vb, [rowv, dv]))
                acc = acc + lwm_s[d] * jnp.minimum(t, 0.0)
            a_buf[pl.ds(c * CHUNK + g * L, L)] = vec + acc
            return _g

        lax.fori_loop(0, CHUNK // L, group_step, 0)
        return carry

    lax.fori_loop(0, NCHUNK, chunk_step, 0)

    def max_step(i, m):
        return jnp.maximum(m, a_buf[pl.ds(i * L, L)])

    m = lax.fori_loop(0, EPW // L, max_step,
                      jnp.full((L,), -jnp.inf, jnp.float32))
    mx_v[...] = m
    pltpu.sync_copy(a_buf, alpha_hbm.at[pl.ds(base, EPW)])
    pltpu.sync_copy(mx_v, maxp_hbm.at[w])


_alpha_kernel = functools.partial(
    pl.kernel,
    out_type=[
        jax.ShapeDtypeStruct((E,), jnp.float32),
        jax.ShapeDtypeStruct((NW, L), jnp.float32),
    ],
    mesh=_mesh,
    compiler_params=pltpu.CompilerParams(needs_layout_passes=False),
    scratch_types=[
        pltpu.VMEM((NCHUNK, CHUNK), jnp.int32),
        pltpu.VMEM((NCHUNK, CHUNK), jnp.int32),
        pltpu.VMEM((2, CHUNK, D), jnp.float32),
        pltpu.VMEM((2, CHUNK, D), jnp.float32),
        pltpu.VMEM((EPW,), jnp.float32),
        pltpu.VMEM((D,), jnp.float32),
        pltpu.VMEM((N,), jnp.float32),
        pltpu.VMEM((N,), jnp.float32),
        pltpu.VMEM((L,), jnp.float32),
        pltpu.SMEM((D,), jnp.float32),
        pltpu.SemaphoreType.DMA,
        pltpu.SemaphoreType.DMA,
        pltpu.SemaphoreType.DMA,
        pltpu.SemaphoreType.DMA,
    ],
)(_alpha_body)


def _sums_body(alpha_hbm, ids_hbm, maxp_hbm,
               ex_hbm, sump_hbm,
               a_buf, ids_buf, mx_all, s_v, s_loc):
    w = _wid()
    base = w * EPW
    pltpu.sync_copy(maxp_hbm, mx_all)
    pltpu.sync_copy(alpha_hbm.at[pl.ds(base, EPW)], a_buf)
    pltpu.sync_copy(ids_hbm.at[pl.ds(base, EPW)], ids_buf)

    def mred(i, m):
        return jnp.maximum(m, mx_all[i, :])

    mvec = lax.fori_loop(0, NW, mred,
                         jnp.full((L,), -jnp.inf, jnp.float32))
    M = jnp.max(mvec)

    def exp_step(i, _):
        a_buf[pl.ds(i * L, L)] = jnp.exp(a_buf[pl.ds(i * L, L)] - M)
        return 0

    lax.fori_loop(0, EPW // L, exp_step, 0)

    def zero_step(b, _):
        s_loc[b] = jnp.float32(0.0)
        return 0

    lax.fori_loop(0, B, zero_step, 0)

    def acc_step(i, carry):
        cur, acc = carry
        ids = ids_buf[pl.ds(i * L, L)]
        xs = a_buf[pl.ds(i * L, L)]
        for j in range(L):
            sid = ids[j]
            x = xs[j]
            pred = sid != cur
            acc = jnp.where(pred, x, acc + x)
            cur = jnp.where(pred, sid, cur)
            s_loc[cur] = acc
        return cur, acc

    cur0 = ids_buf[pl.ds(0, L)][0]
    lax.fori_loop(0, EPW // L, acc_step, (cur0, jnp.float32(0.0)))

    lanes = jnp.arange(L, dtype=jnp.int32)

    def pack_step(jv, _):
        vec = jnp.zeros((L,), jnp.float32)
        for j in range(L):
            vec = jnp.where(lanes == j, s_loc[jv * L + j], vec)
        s_v[pl.ds(jv * L, L)] = vec
        return 0

    lax.fori_loop(0, B // L, pack_step, 0)

    pltpu.sync_copy(a_buf, ex_hbm.at[pl.ds(base, EPW)])
    pltpu.sync_copy(s_v, sump_hbm.at[w])


_sums_kernel = functools.partial(
    pl.kernel,
    out_type=[
        jax.ShapeDtypeStruct((E,), jnp.float32),
        jax.ShapeDtypeStruct((NW, B), jnp.float32),
    ],
    mesh=_mesh,
    compiler_params=pltpu.CompilerParams(needs_layout_passes=False),
    scratch_types=[
        pltpu.VMEM((EPW,), jnp.float32),
        pltpu.VMEM((EPW,), jnp.int32),
        pltpu.VMEM((NW, L), jnp.float32),
        pltpu.VMEM((B,), jnp.float32),
        pltpu.SMEM((B,), jnp.float32),
    ],
)(_sums_body)


def _norm_body(ex_hbm, ids_hbm, sump_hbm,
               out_hbm,
               ex_buf, ids_buf, sp_buf, s_buf):
    w = _wid()
    base = w * EPW
    pltpu.sync_copy(sump_hbm, sp_buf)
    pltpu.sync_copy(ex_hbm.at[pl.ds(base, EPW)], ex_buf)
    pltpu.sync_copy(ids_hbm.at[pl.ds(base, EPW)], ids_buf)

    def comb_step(j, _):
        def row_step(r, acc):
            return acc + sp_buf[r, pl.ds(j * L, L)]

        s_buf[pl.ds(j * L, L)] = lax.fori_loop(
            0, NW, row_step, jnp.zeros((L,), jnp.float32))
        return 0

    lax.fori_loop(0, B // L, comb_step, 0)

    def norm_step(i, _):
        ids = ids_buf[pl.ds(i * L, L)]
        s = plsc.load_gather(s_buf, [ids])
        ex_buf[pl.ds(i * L, L)] = ex_buf[pl.ds(i * L, L)] / s
        return 0

    lax.fori_loop(0, EPW // L, norm_step, 0)
    pltpu.sync_copy(ex_buf, out_hbm.at[pl.ds(base, EPW)])


_norm_kernel = functools.partial(
    pl.kernel,
    out_type=jax.ShapeDtypeStruct((E,), jnp.float32),
    mesh=_mesh,
    compiler_params=pltpu.CompilerParams(needs_layout_passes=False),
    scratch_types=[
        pltpu.VMEM((EPW,), jnp.float32),
        pltpu.VMEM((EPW,), jnp.int32),
        pltpu.VMEM((NW, B), jnp.float32),
        pltpu.VMEM((B,), jnp.float32),
    ],
)(_norm_body)


def kernel(x_j, x_i, edge_index, edge_index_batch, w_j, w_i, bias,
           prelu_w, lin_w, lin_b):
    src = edge_index[0].reshape(NW, NCHUNK, CHUNK)
    dst = edge_index[1].reshape(NW, NCHUNK, CHUNK)
    bias2d = bias.reshape(1, D)
    lw2d = lin_w.reshape(1, D)
    lwm = ((prelu_w[0] - 1.0) * lin_w).reshape(D)

    xj, xib, sj, si = _project(x_j, x_i, w_j, w_i, bias2d, lw2d)
    alpha, maxp = _alpha_kernel(xj, xib, src, dst, lwm,
                                sj.reshape(N), si.reshape(N))
    ex, sump = _sums_kernel(alpha, edge_index_batch, maxp)
    return _norm_kernel(ex, edge_index_batch, sump)


# speedup vs baseline: 2.8865x; 2.8865x over previous
import functools

import jax
import jax.numpy as jnp
from jax import lax
from jax.experimental import pallas as pl
from jax.experimental.pallas import tpu as pltpu
from jax.experimental.pallas import tpu_sc as plsc

N = 10000
E = 320000
D = 128
B = 1024

NC = 2
NS = 16
L = 16
NW = NC * NS
EPW = E // NW
CHUNK = 80
NCHUNK = EPW // CHUNK
KV = D // L

_mesh = plsc.VectorSubcoreMesh(core_axis_name="c", subcore_axis_name="s")


def _wid():
    return lax.axis_index("s") * NC + lax.axis_index("c")


def _proj_body(xj_ref, xi_ref, wj_ref, wi_ref, bias_ref, lw_ref,
               oj_ref, oi_ref, sj_ref, si_ref):
    oj = jnp.dot(xj_ref[...], wj_ref[...], preferred_element_type=jnp.float32)
    oi = jnp.dot(xi_ref[...], wi_ref[...],
                 preferred_element_type=jnp.float32) + bias_ref[...]
    oj_ref[...] = oj
    oi_ref[...] = oi
    lw_col = lw_ref[...].reshape(D, 1)
    sj_ref[...] = jnp.dot(oj, lw_col, preferred_element_type=jnp.float32)
    si_ref[...] = jnp.dot(oi, lw_col, preferred_element_type=jnp.float32)


def _project(x_j, x_i, w_j, w_i, bias2d, lw2d):
    blk = 2000
    grid = (N // blk,)
    return pl.pallas_call(
        _proj_body,
        grid=grid,
        in_specs=[
            pl.BlockSpec((blk, D), lambda i: (i, 0)),
            pl.BlockSpec((blk, D), lambda i: (i, 0)),
            pl.BlockSpec((D, D), lambda i: (0, 0)),
            pl.BlockSpec((D, D), lambda i: (0, 0)),
            pl.BlockSpec((1, D), lambda i: (0, 0)),
            pl.BlockSpec((1, D), lambda i: (0, 0)),
        ],
        out_specs=[
            pl.BlockSpec((blk, D), lambda i: (i, 0)),
            pl.BlockSpec((blk, D), lambda i: (i, 0)),
            pl.BlockSpec((blk, 1), lambda i: (i, 0)),
            pl.BlockSpec((blk, 1), lambda i: (i, 0)),
        ],
        out_shape=[
            jax.ShapeDtypeStruct((N, D), jnp.float32),
            jax.ShapeDtypeStruct((N, D), jnp.float32),
            jax.ShapeDtypeStruct((N, 1), jnp.float32),
            jax.ShapeDtypeStruct((N, 1), jnp.float32),
        ],
    )(x_j, x_i, w_j, w_i, bias2d, lw2d)


def _alpha_body(xj_hbm, xib_hbm, srcr_hbm, dstr_hbm, lwm_hbm, sj_hbm, si_hbm,
                alpha_hbm, maxp_hbm,
                src_all, dst_all, u_rows, v_rows, a_buf, lwm_v, sj_v, si_v,
                mx_v, sem_u0, sem_v0, sem_u1, sem_v1):
    w = _wid()
    base = w * EPW
    pltpu.sync_copy(lwm_hbm, lwm_v)
    pltpu.sync_copy(sj_hbm, sj_v)
    pltpu.sync_copy(si_hbm, si_v)
    pltpu.sync_copy(srcr_hbm.at[w], src_all)
    pltpu.sync_copy(dstr_hbm.at[w], dst_all)
    lwm_regs = [lwm_v[pl.ds(k * L, L)] for k in range(KV)]
    lanes = jnp.arange(L, dtype=jnp.int32)
    sems = [(sem_u0, sem_v0), (sem_u1, sem_v1)]

    def fetch(cc, bb):
        su, sv = sems[bb]
        pltpu.async_copy(xj_hbm.at[src_all.at[cc]], u_rows.at[bb], su)
        pltpu.async_copy(xib_hbm.at[dst_all.at[cc]], v_rows.at[bb], sv)

    def drain(cc, bb):
        su, sv = sems[bb]
        pltpu.make_async_copy(
            xj_hbm.at[src_all.at[cc]], u_rows.at[bb], su).wait()
        pltpu.make_async_copy(
            xib_hbm.at[dst_all.at[cc]], v_rows.at[bb], sv).wait()

    fetch(0, 0)

    def chunk_step(c, carry):
        b = lax.rem(c, 2)

        @pl.when(jnp.logical_and(c + 1 < NCHUNK, b == 0))
        def _():
            fetch(c + 1, 1)

        @pl.when(jnp.logical_and(c + 1 < NCHUNK, b == 1))
        def _():
            fetch(c + 1, 0)

        @pl.when(b == 0)
        def _():
            drain(c, 0)

        @pl.when(b == 1)
        def _():
            drain(c, 1)

        def group_step(g, _g):
            e0 = g * L
            sidx = src_all[c, pl.ds(e0, L)]
            didx = dst_all[c, pl.ds(e0, L)]
            vec = (plsc.load_gather(sj_v, [sidx])
                   + plsc.load_gather(si_v, [didx]))
            mvec = jnp.zeros((L,), jnp.float32)
            for j in range(L):
                e = e0 + j
                acc = jnp.zeros((L,), jnp.float32)
                for k in range(KV):
                    t = (u_rows[b, e, pl.ds(k * L, L)]
                         + v_rows[b, e, pl.ds(k * L, L)])
                    acc = acc + lwm_regs[k] * jnp.minimum(t, 0.0)
                mvec = jnp.where(lanes == j, jnp.sum(acc), mvec)
            a_buf[pl.ds(c * CHUNK + g * L, L)] = vec + mvec
            return _g

        lax.fori_loop(0, CHUNK // L, group_step, 0)
        return carry

    lax.fori_loop(0, NCHUNK, chunk_step, 0)

    def max_step(i, m):
        return jnp.maximum(m, a_buf[pl.ds(i * L, L)])

    m = lax.fori_loop(0, EPW // L, max_step,
                      jnp.full((L,), -jnp.inf, jnp.float32))
    mx_v[...] = m
    pltpu.sync_copy(a_buf, alpha_hbm.at[pl.ds(base, EPW)])
    pltpu.sync_copy(mx_v, maxp_hbm.at[w])


_alpha_kernel = functools.partial(
    pl.kernel,
    out_type=[
        jax.ShapeDtypeStruct((E,), jnp.float32),
        jax.ShapeDtypeStruct((NW, L), jnp.float32),
    ],
    mesh=_mesh,
    compiler_params=pltpu.CompilerParams(needs_layout_passes=False),
    scratch_types=[
        pltpu.VMEM((NCHUNK, CHUNK), jnp.int32),
        pltpu.VMEM((NCHUNK, CHUNK), jnp.int32),
        pltpu.VMEM((2, CHUNK, D), jnp.float32),
        pltpu.VMEM((2, CHUNK, D), jnp.float32),
        pltpu.VMEM((EPW,), jnp.float32),
        pltpu.VMEM((D,), jnp.float32),
        pltpu.VMEM((N,), jnp.float32),
        pltpu.VMEM((N,), jnp.float32),
        pltpu.VMEM((L,), jnp.float32),
        pltpu.SemaphoreType.DMA,
        pltpu.SemaphoreType.DMA,
        pltpu.SemaphoreType.DMA,
        pltpu.SemaphoreType.DMA,
    ],
)(_alpha_body)


def _sums_body(alpha_hbm, ids_hbm, maxp_hbm,
               ex_hbm, sump_hbm,
               a_buf, ids_buf, mx_all, s_v, s_loc):
    w = _wid()
    base = w * EPW
    pltpu.sync_copy(maxp_hbm, mx_all)
    pltpu.sync_copy(alpha_hbm.at[pl.ds(base, EPW)], a_buf)
    pltpu.sync_copy(ids_hbm.at[pl.ds(base, EPW)], ids_buf)

    def mred(i, m):
        return jnp.maximum(m, mx_all[i, :])

    mvec = lax.fori_loop(0, NW, mred,
                         jnp.full((L,), -jnp.inf, jnp.float32))
    M = jnp.max(mvec)

    def exp_step(i, _):
        a_buf[pl.ds(i * L, L)] = jnp.exp(a_buf[pl.ds(i * L, L)] - M)
        return 0

    lax.fori_loop(0, EPW // L, exp_step, 0)

    def zero_step(b, _):
        s_loc[b] = jnp.float32(0.0)
        return 0

    lax.fori_loop(0, B, zero_step, 0)

    def acc_step(i, carry):
        cur, acc = carry
        ids = ids_buf[pl.ds(i * L, L)]
        xs = a_buf[pl.ds(i * L, L)]
        for j in range(L):
            sid = ids[j]
            x = xs[j]
            pred = sid != cur
            acc = jnp.where(pred, x, acc + x)
            cur = jnp.where(pred, sid, cur)
            s_loc[cur] = acc
        return cur, acc

    cur0 = ids_buf[pl.ds(0, L)][0]
    lax.fori_loop(0, EPW // L, acc_step, (cur0, jnp.float32(0.0)))

    lanes = jnp.arange(L, dtype=jnp.int32)

    def pack_step(jv, _):
        vec = jnp.zeros((L,), jnp.float32)
        for j in range(L):
            vec = jnp.where(lanes == j, s_loc[jv * L + j], vec)
        s_v[pl.ds(jv * L, L)] = vec
        return 0

    lax.fori_loop(0, B // L, pack_step, 0)

    pltpu.sync_copy(a_buf, ex_hbm.at[pl.ds(base, EPW)])
    pltpu.sync_copy(s_v, sump_hbm.at[w])


_sums_kernel = functools.partial(
    pl.kernel,
    out_type=[
        jax.ShapeDtypeStruct((E,), jnp.float32),
        jax.ShapeDtypeStruct((NW, B), jnp.float32),
    ],
    mesh=_mesh,
    compiler_params=pltpu.CompilerParams(needs_layout_passes=False),
    scratch_types=[
        pltpu.VMEM((EPW,), jnp.float32),
        pltpu.VMEM((EPW,), jnp.int32),
        pltpu.VMEM((NW, L), jnp.float32),
        pltpu.VMEM((B,), jnp.float32),
        pltpu.SMEM((B,), jnp.float32),
    ],
)(_sums_body)


def _norm_body(ex_hbm, ids_hbm, sump_hbm,
               out_hbm,
               ex_buf, ids_buf, sp_buf, s_buf):
    w = _wid()
    base = w * EPW
    pltpu.sync_copy(sump_hbm, sp_buf)
    pltpu.sync_copy(ex_hbm.at[pl.ds(base, EPW)], ex_buf)
    pltpu.sync_copy(ids_hbm.at[pl.ds(base, EPW)], ids_buf)

    def comb_step(j, _):
        def row_step(r, acc):
            return acc + sp_buf[r, pl.ds(j * L, L)]

        s_buf[pl.ds(j * L, L)] = lax.fori_loop(
            0, NW, row_step, jnp.zeros((L,), jnp.float32))
        return 0

    lax.fori_loop(0, B // L, comb_step, 0)

    def norm_step(i, _):
        ids = ids_buf[pl.ds(i * L, L)]
        s = plsc.load_gather(s_buf, [ids])
        ex_buf[pl.ds(i * L, L)] = ex_buf[pl.ds(i * L, L)] / s
        return 0

    lax.fori_loop(0, EPW // L, norm_step, 0)
    pltpu.sync_copy(ex_buf, out_hbm.at[pl.ds(base, EPW)])


_norm_kernel = functools.partial(
    pl.kernel,
    out_type=jax.ShapeDtypeStruct((E,), jnp.float32),
    mesh=_mesh,
    compiler_params=pltpu.CompilerParams(needs_layout_passes=False),
    scratch_types=[
        pltpu.VMEM((EPW,), jnp.float32),
        pltpu.VMEM((EPW,), jnp.int32),
        pltpu.VMEM((NW, B), jnp.float32),
        pltpu.VMEM((B,), jnp.float32),
    ],
)(_norm_body)


def kernel(x_j, x_i, edge_index, edge_index_batch, w_j, w_i, bias,
           prelu_w, lin_w, lin_b):
    src = edge_index[0].reshape(NW, NCHUNK, CHUNK)
    dst = edge_index[1].reshape(NW, NCHUNK, CHUNK)
    bias2d = bias.reshape(1, D)
    lw2d = lin_w.reshape(1, D)
    lwm = ((prelu_w[0] - 1.0) * lin_w).reshape(D)

    xj, xib, sj, si = _project(x_j, x_i, w_j, w_i, bias2d, lw2d)
    alpha, maxp = _alpha_kernel(xj, xib, src, dst, lwm,
                                sj.reshape(N), si.reshape(N))
    ex, sump = _sums_kernel(alpha, edge_index_batch, maxp)
    return _norm_kernel(ex, edge_index_batch, sump)


# speedup vs baseline: 4.9437x; 1.7127x over previous
import functools

import jax
import jax.numpy as jnp
from jax import lax
from jax.experimental import pallas as pl
from jax.experimental.pallas import tpu as pltpu
from jax.experimental.pallas import tpu_sc as plsc

N = 10000
E = 320000
D = 128
B = 1024

NC = 2
NS = 16
L = 16
NW = NC * NS
EPW = E // NW
CHUNK = 80
NCHUNK = EPW // CHUNK
KV = D // L

_mesh = plsc.VectorSubcoreMesh(core_axis_name="c", subcore_axis_name="s")


def _wid():
    return lax.axis_index("s") * NC + lax.axis_index("c")


def _proj_body(xj_ref, xi_ref, wj_ref, wi_ref, bias_ref, lw_ref,
               oj_ref, oi_ref, sj_ref, si_ref):
    oj = jnp.dot(xj_ref[...], wj_ref[...], preferred_element_type=jnp.float32)
    oi = jnp.dot(xi_ref[...], wi_ref[...],
                 preferred_element_type=jnp.float32) + bias_ref[...]
    oj_ref[...] = oj
    oi_ref[...] = oi
    lw_col = lw_ref[...].reshape(D, 1)
    sj_ref[...] = jnp.dot(oj, lw_col, preferred_element_type=jnp.float32)
    si_ref[...] = jnp.dot(oi, lw_col, preferred_element_type=jnp.float32)


def _project(x_j, x_i, w_j, w_i, bias2d, lw2d):
    blk = 2000
    grid = (N // blk,)
    return pl.pallas_call(
        _proj_body,
        grid=grid,
        in_specs=[
            pl.BlockSpec((blk, D), lambda i: (i, 0)),
            pl.BlockSpec((blk, D), lambda i: (i, 0)),
            pl.BlockSpec((D, D), lambda i: (0, 0)),
            pl.BlockSpec((D, D), lambda i: (0, 0)),
            pl.BlockSpec((1, D), lambda i: (0, 0)),
            pl.BlockSpec((1, D), lambda i: (0, 0)),
        ],
        out_specs=[
            pl.BlockSpec((blk, D), lambda i: (i, 0)),
            pl.BlockSpec((blk, D), lambda i: (i, 0)),
            pl.BlockSpec((blk, 1), lambda i: (i, 0)),
            pl.BlockSpec((blk, 1), lambda i: (i, 0)),
        ],
        out_shape=[
            jax.ShapeDtypeStruct((N, D), jnp.float32),
            jax.ShapeDtypeStruct((N, D), jnp.float32),
            jax.ShapeDtypeStruct((N, 1), jnp.float32),
            jax.ShapeDtypeStruct((N, 1), jnp.float32),
        ],
    )(x_j, x_i, w_j, w_i, bias2d, lw2d)


def _alpha_body(xj_hbm, xib_hbm, srcr_hbm, dstr_hbm, lw_hbm, lwp_hbm,
                alpha_hbm, maxp_hbm,
                src_all, dst_all, u_rows, v_rows, a_buf, lw_v, lwp_v,
                mx_v, sem_u0, sem_v0, sem_u1, sem_v1):
    w = _wid()
    base = w * EPW
    pltpu.sync_copy(lw_hbm, lw_v)
    pltpu.sync_copy(lwp_hbm, lwp_v)
    pltpu.sync_copy(srcr_hbm.at[w], src_all)
    pltpu.sync_copy(dstr_hbm.at[w], dst_all)
    lw_regs = [lw_v[pl.ds(k * L, L)] for k in range(KV)]
    lwp_regs = [lwp_v[pl.ds(k * L, L)] for k in range(KV)]
    lanes = jnp.arange(L, dtype=jnp.int32)
    sems = [(sem_u0, sem_v0), (sem_u1, sem_v1)]

    def fetch(cc, bb):
        su, sv = sems[bb]
        pltpu.async_copy(xj_hbm.at[src_all.at[cc]], u_rows.at[bb], su)
        pltpu.async_copy(xib_hbm.at[dst_all.at[cc]], v_rows.at[bb], sv)

    def drain(cc, bb):
        su, sv = sems[bb]
        pltpu.make_async_copy(
            xj_hbm.at[src_all.at[cc]], u_rows.at[bb], su).wait()
        pltpu.make_async_copy(
            xib_hbm.at[dst_all.at[cc]], v_rows.at[bb], sv).wait()

    fetch(0, 0)

    def chunk_step(c, carry):
        b = lax.rem(c, 2)

        @pl.when(jnp.logical_and(c + 1 < NCHUNK, b == 0))
        def _():
            fetch(c + 1, 1)

        @pl.when(jnp.logical_and(c + 1 < NCHUNK, b == 1))
        def _():
            fetch(c + 1, 0)

        @pl.when(b == 0)
        def _():
            drain(c, 0)

        @pl.when(b == 1)
        def _():
            drain(c, 1)

        def group_step(g, _g):
            e0 = g * L
            vec = jnp.zeros((L,), jnp.float32)
            for j in range(L):
                e = e0 + j
                acc1 = jnp.zeros((L,), jnp.float32)
                acc2 = jnp.zeros((L,), jnp.float32)
                for k in range(KV):
                    t = (u_rows[b, e, pl.ds(k * L, L)]
                         + v_rows[b, e, pl.ds(k * L, L)])
                    acc1 = acc1 + lw_regs[k] * jnp.maximum(t, 0.0)
                    acc2 = acc2 + lwp_regs[k] * jnp.minimum(t, 0.0)
                vec = jnp.where(lanes == j, jnp.sum(acc1 + acc2), vec)
            a_buf[pl.ds(c * CHUNK + g * L, L)] = vec
            return _g

        lax.fori_loop(0, CHUNK // L, group_step, 0)
        return carry

    lax.fori_loop(0, NCHUNK, chunk_step, 0)

    def max_step(i, m):
        return jnp.maximum(m, a_buf[pl.ds(i * L, L)])

    m = lax.fori_loop(0, EPW // L, max_step,
                      jnp.full((L,), -jnp.inf, jnp.float32))
    mx_v[...] = m
    pltpu.sync_copy(a_buf, alpha_hbm.at[pl.ds(base, EPW)])
    pltpu.sync_copy(mx_v, maxp_hbm.at[w])


_alpha_kernel = functools.partial(
    pl.kernel,
    out_type=[
        jax.ShapeDtypeStruct((E,), jnp.float32),
        jax.ShapeDtypeStruct((NW, L), jnp.float32),
    ],
    mesh=_mesh,
    compiler_params=pltpu.CompilerParams(needs_layout_passes=False),
    scratch_types=[
        pltpu.VMEM((NCHUNK, CHUNK), jnp.int32),
        pltpu.VMEM((NCHUNK, CHUNK), jnp.int32),
        pltpu.VMEM((2, CHUNK, D), jnp.float32),
        pltpu.VMEM((2, CHUNK, D), jnp.float32),
        pltpu.VMEM((EPW,), jnp.float32),
        pltpu.VMEM((D,), jnp.float32),
        pltpu.VMEM((D,), jnp.float32),
        pltpu.VMEM((L,), jnp.float32),
        pltpu.SemaphoreType.DMA,
        pltpu.SemaphoreType.DMA,
        pltpu.SemaphoreType.DMA,
        pltpu.SemaphoreType.DMA,
    ],
)(_alpha_body)


def _sums_body(alpha_hbm, ids_hbm, maxp_hbm,
               ex_hbm, sump_hbm,
               a_buf, ids_buf, mx_all, s_v, s_loc):
    w = _wid()
    base = w * EPW
    pltpu.sync_copy(maxp_hbm, mx_all)
    pltpu.sync_copy(alpha_hbm.at[pl.ds(base, EPW)], a_buf)
    pltpu.sync_copy(ids_hbm.at[pl.ds(base, EPW)], ids_buf)

    def mred(i, m):
        return jnp.maximum(m, mx_all[i, :])

    mvec = lax.fori_loop(0, NW, mred,
                         jnp.full((L,), -jnp.inf, jnp.float32))
    M = jnp.max(mvec)

    def exp_step(i, _):
        a_buf[pl.ds(i * L, L)] = jnp.exp(a_buf[pl.ds(i * L, L)] - M)
        return 0

    lax.fori_loop(0, EPW // L, exp_step, 0)

    def zero_step(b, _):
        s_loc[b] = jnp.float32(0.0)
        return 0

    lax.fori_loop(0, B, zero_step, 0)

    def acc_step(i, carry):
        cur, acc = carry
        ids = ids_buf[pl.ds(i * L, L)]
        xs = a_buf[pl.ds(i * L, L)]
        for j in range(L):
            sid = ids[j]
            x = xs[j]
            pred = sid != cur
            acc = jnp.where(pred, x, acc + x)
            cur = jnp.where(pred, sid, cur)
            s_loc[cur] = acc
        return cur, acc

    cur0 = ids_buf[pl.ds(0, L)][0]
    lax.fori_loop(0, EPW // L, acc_step, (cur0, jnp.float32(0.0)))

    lanes = jnp.arange(L, dtype=jnp.int32)

    def pack_step(jv, _):
        vec = jnp.zeros((L,), jnp.float32)
        for j in range(L):
            vec = jnp.where(lanes == j, s_loc[jv * L + j], vec)
        s_v[pl.ds(jv * L, L)] = vec
        return 0

    lax.fori_loop(0, B // L, pack_step, 0)

    pltpu.sync_copy(a_buf, ex_hbm.at[pl.ds(base, EPW)])
    pltpu.sync_copy(s_v, sump_hbm.at[w])


_sums_kernel = functools.partial(
    pl.kernel,
    out_type=[
        jax.ShapeDtypeStruct((E,), jnp.float32),
        jax.ShapeDtypeStruct((NW, B), jnp.float32),
    ],
    mesh=_mesh,
    compiler_params=pltpu.CompilerParams(needs_layout_passes=False),
    scratch_types=[
        pltpu.VMEM((EPW,), jnp.float32),
        pltpu.VMEM((EPW,), jnp.int32),
        pltpu.VMEM((NW, L), jnp.float32),
        pltpu.VMEM((B,), jnp.float32),
        pltpu.SMEM((B,), jnp.float32),
    ],
)(_sums_body)


def _norm_body(ex_hbm, ids_hbm, sump_hbm,
               out_hbm,
               ex_buf, ids_buf, sp_buf, s_buf):
    w = _wid()
    base = w * EPW
    pltpu.sync_copy(sump_hbm, sp_buf)
    pltpu.sync_copy(ex_hbm.at[pl.ds(base, EPW)], ex_buf)
    pltpu.sync_copy(ids_hbm.at[pl.ds(base, EPW)], ids_buf)

    def comb_step(j, _):
        def row_step(r, acc):
            return acc + sp_buf[r, pl.ds(j * L, L)]

        s_buf[pl.ds(j * L, L)] = lax.fori_loop(
            0, NW, row_step, jnp.zeros((L,), jnp.float32))
        return 0

    lax.fori_loop(0, B // L, comb_step, 0)

    def norm_step(i, _):
        ids = ids_buf[pl.ds(i * L, L)]
        s = plsc.load_gather(s_buf, [ids])
        ex_buf[pl.ds(i * L, L)] = ex_buf[pl.ds(i * L, L)] / s
        return 0

    lax.fori_loop(0, EPW // L, norm_step, 0)
    pltpu.sync_copy(ex_buf, out_hbm.at[pl.ds(base, EPW)])


_norm_kernel = functools.partial(
    pl.kernel,
    out_type=jax.ShapeDtypeStruct((E,), jnp.float32),
    mesh=_mesh,
    compiler_params=pltpu.CompilerParams(needs_layout_passes=False),
    scratch_types=[
        pltpu.VMEM((EPW,), jnp.float32),
        pltpu.VMEM((EPW,), jnp.int32),
        pltpu.VMEM((NW, B), jnp.float32),
        pltpu.VMEM((B,), jnp.float32),
    ],
)(_norm_body)


def kernel(x_j, x_i, edge_index, edge_index_batch, w_j, w_i, bias,
           prelu_w, lin_w, lin_b):
    src = edge_index[0].reshape(NW, NCHUNK, CHUNK)
    dst = edge_index[1].reshape(NW, NCHUNK, CHUNK)
    bias2d = bias.reshape(1, D)
    lw2d = lin_w.reshape(1, D)
    lw = lin_w.reshape(D)
    lwp = (prelu_w[0] * lin_w).reshape(D)

    xj, xib, sj, si = _project(x_j, x_i, w_j, w_i, bias2d, lw2d)
    alpha, maxp = _alpha_kernel(xj, xib, src, dst, lw, lwp)
    ex, sump = _sums_kernel(alpha, edge_index_batch, maxp)
    return _norm_kernel(ex, edge_index_batch, sump)


# speedup vs baseline: 5.4139x; 1.0951x over previous
import functools

import jax
import jax.numpy as jnp
from jax import lax
from jax.experimental import pallas as pl
from jax.experimental.pallas import tpu as pltpu
from jax.experimental.pallas import tpu_sc as plsc

N = 10000
E = 320000
D = 128
B = 1024

NC = 2
NS = 16
L = 16
NW = NC * NS
EPW = E // NW
CHUNK = 80
NCHUNK = EPW // CHUNK
KV = D // L

_mesh = plsc.VectorSubcoreMesh(core_axis_name="c", subcore_axis_name="s")


def _wid():
    return lax.axis_index("s") * NC + lax.axis_index("c")


def _proj_body(xj_ref, xi_ref, wj_ref, wi_ref, bias_ref, lw_ref,
               oj_ref, oi_ref, sj_ref, si_ref):
    oj = jnp.dot(xj_ref[...], wj_ref[...], preferred_element_type=jnp.float32)
    oi = jnp.dot(xi_ref[...], wi_ref[...],
                 preferred_element_type=jnp.float32) + bias_ref[...]
    oj_ref[...] = oj
    oi_ref[...] = oi
    lw_col = lw_ref[...].reshape(D, 1)
    sj_ref[...] = jnp.dot(oj, lw_col, preferred_element_type=jnp.float32)
    si_ref[...] = jnp.dot(oi, lw_col, preferred_element_type=jnp.float32)


def _project(x_j, x_i, w_j, w_i, bias2d, lw2d):
    blk = 2000
    grid = (N // blk,)
    return pl.pallas_call(
        _proj_body,
        grid=grid,
        in_specs=[
            pl.BlockSpec((blk, D), lambda i: (i, 0)),
            pl.BlockSpec((blk, D), lambda i: (i, 0)),
            pl.BlockSpec((D, D), lambda i: (0, 0)),
            pl.BlockSpec((D, D), lambda i: (0, 0)),
            pl.BlockSpec((1, D), lambda i: (0, 0)),
            pl.BlockSpec((1, D), lambda i: (0, 0)),
        ],
        out_specs=[
            pl.BlockSpec((blk, D), lambda i: (i, 0)),
            pl.BlockSpec((blk, D), lambda i: (i, 0)),
            pl.BlockSpec((blk, 1), lambda i: (i, 0)),
            pl.BlockSpec((blk, 1), lambda i: (i, 0)),
        ],
        out_shape=[
            jax.ShapeDtypeStruct((N, D), jnp.float32),
            jax.ShapeDtypeStruct((N, D), jnp.float32),
            jax.ShapeDtypeStruct((N, 1), jnp.float32),
            jax.ShapeDtypeStruct((N, 1), jnp.float32),
        ],
    )(x_j, x_i, w_j, w_i, bias2d, lw2d)


def _alpha_body(xj_hbm, xib_hbm, srcr_hbm, dstr_hbm, lw_hbm, lwp_hbm,
                ids_hbm,
                ex_hbm, sump_hbm,
                src_all, dst_all, u_rows, v_rows, a_buf, lw_v, lwp_v,
                ids_buf, s_v, s_loc, sem_u0, sem_v0, sem_u1, sem_v1):
    w = _wid()
    base = w * EPW
    pltpu.sync_copy(lw_hbm, lw_v)
    pltpu.sync_copy(lwp_hbm, lwp_v)
    pltpu.sync_copy(srcr_hbm.at[w], src_all)
    pltpu.sync_copy(dstr_hbm.at[w], dst_all)
    pltpu.sync_copy(ids_hbm.at[pl.ds(base, EPW)], ids_buf)
    lw_regs = [lw_v[pl.ds(k * L, L)] for k in range(KV)]
    lwp_regs = [lwp_v[pl.ds(k * L, L)] for k in range(KV)]
    lanes = jnp.arange(L, dtype=jnp.int32)
    sems = [(sem_u0, sem_v0), (sem_u1, sem_v1)]

    def zero_step(bq, _):
        s_loc[bq] = jnp.float32(0.0)
        return 0

    lax.fori_loop(0, B, zero_step, 0)

    def fetch(cc, bb):
        su, sv = sems[bb]
        pltpu.async_copy(xj_hbm.at[src_all.at[cc]], u_rows.at[bb], su)
        pltpu.async_copy(xib_hbm.at[dst_all.at[cc]], v_rows.at[bb], sv)

    def drain(cc, bb):
        su, sv = sems[bb]
        pltpu.make_async_copy(
            xj_hbm.at[src_all.at[cc]], u_rows.at[bb], su).wait()
        pltpu.make_async_copy(
            xib_hbm.at[dst_all.at[cc]], v_rows.at[bb], sv).wait()

    fetch(0, 0)

    def chunk_step(c, carry):
        b = lax.rem(c, 2)

        @pl.when(jnp.logical_and(c + 1 < NCHUNK, b == 0))
        def _():
            fetch(c + 1, 1)

        @pl.when(jnp.logical_and(c + 1 < NCHUNK, b == 1))
        def _():
            fetch(c + 1, 0)

        @pl.when(b == 0)
        def _():
            drain(c, 0)

        @pl.when(b == 1)
        def _():
            drain(c, 1)

        def group_step(g, _g):
            e0 = g * L
            vec = jnp.zeros((L,), jnp.float32)
            for j in range(L):
                e = e0 + j
                acc1 = jnp.zeros((L,), jnp.float32)
                acc2 = jnp.zeros((L,), jnp.float32)
                for k in range(KV):
                    t = (u_rows[b, e, pl.ds(k * L, L)]
                         + v_rows[b, e, pl.ds(k * L, L)])
                    acc1 = acc1 + lw_regs[k] * jnp.maximum(t, 0.0)
                    acc2 = acc2 + lwp_regs[k] * jnp.minimum(t, 0.0)
                vec = jnp.where(lanes == j, jnp.sum(acc1 + acc2), vec)
            a_buf[pl.ds(c * CHUNK + g * L, L)] = jnp.exp(vec)
            return _g

        lax.fori_loop(0, CHUNK // L, group_step, 0)

        def seg_step(q, carry2):
            cur, acc = carry2
            off = c * CHUNK + q * L
            ids = ids_buf[pl.ds(off, L)]
            xs = a_buf[pl.ds(off, L)]
            for j in range(L):
                sid = ids[j]
                x = xs[j]
                pred = sid != cur
                acc = jnp.where(pred, x, acc + x)
                cur = jnp.where(pred, sid, cur)
                s_loc[cur] = acc
            return cur, acc

        return lax.fori_loop(0, CHUNK // L, seg_step, carry)

    cur0 = ids_buf[pl.ds(0, L)][0]
    lax.fori_loop(0, NCHUNK, chunk_step, (cur0, jnp.float32(0.0)))

    def pack_step(jv, _):
        pvec = jnp.zeros((L,), jnp.float32)
        for j in range(L):
            pvec = jnp.where(lanes == j, s_loc[jv * L + j], pvec)
        s_v[pl.ds(jv * L, L)] = pvec
        return 0

    lax.fori_loop(0, B // L, pack_step, 0)

    pltpu.sync_copy(a_buf, ex_hbm.at[pl.ds(base, EPW)])
    pltpu.sync_copy(s_v, sump_hbm.at[w])


_alpha_kernel = functools.partial(
    pl.kernel,
    out_type=[
        jax.ShapeDtypeStruct((E,), jnp.float32),
        jax.ShapeDtypeStruct((NW, B), jnp.float32),
    ],
    mesh=_mesh,
    compiler_params=pltpu.CompilerParams(needs_layout_passes=False),
    scratch_types=[
        pltpu.VMEM((NCHUNK, CHUNK), jnp.int32),
        pltpu.VMEM((NCHUNK, CHUNK), jnp.int32),
        pltpu.VMEM((2, CHUNK, D), jnp.float32),
        pltpu.VMEM((2, CHUNK, D), jnp.float32),
        pltpu.VMEM((EPW,), jnp.float32),
        pltpu.VMEM((D,), jnp.float32),
        pltpu.VMEM((D,), jnp.float32),
        pltpu.VMEM((EPW,), jnp.int32),
        pltpu.VMEM((B,), jnp.float32),
        pltpu.SMEM((B,), jnp.float32),
        pltpu.SemaphoreType.DMA,
        pltpu.SemaphoreType.DMA,
        pltpu.SemaphoreType.DMA,
        pltpu.SemaphoreType.DMA,
    ],
)(_alpha_body)


def _sums_body(alpha_hbm, ids_hbm, maxp_hbm,
               ex_hbm, sump_hbm,
               a_buf, ids_buf, mx_all, s_v, s_loc):
    w = _wid()
    base = w * EPW
    pltpu.sync_copy(maxp_hbm, mx_all)
    pltpu.sync_copy(alpha_hbm.at[pl.ds(base, EPW)], a_buf)
    pltpu.sync_copy(ids_hbm.at[pl.ds(base, EPW)], ids_buf)

    def mred(i, m):
        return jnp.maximum(m, mx_all[i, :])

    mvec = lax.fori_loop(0, NW, mred,
                         jnp.full((L,), -jnp.inf, jnp.float32))
    M = jnp.max(mvec)

    def exp_step(i, _):
        a_buf[pl.ds(i * L, L)] = jnp.exp(a_buf[pl.ds(i * L, L)] - M)
        return 0

    lax.fori_loop(0, EPW // L, exp_step, 0)

    def zero_step(b, _):
        s_loc[b] = jnp.float32(0.0)
        return 0

    lax.fori_loop(0, B, zero_step, 0)

    def acc_step(i, carry):
        cur, acc = carry
        ids = ids_buf[pl.ds(i * L, L)]
        xs = a_buf[pl.ds(i * L, L)]
        for j in range(L):
            sid = ids[j]
            x = xs[j]
            pred = sid != cur
            acc = jnp.where(pred, x, acc + x)
            cur = jnp.where(pred, sid, cur)
            s_loc[cur] = acc
        return cur, acc

    cur0 = ids_buf[pl.ds(0, L)][0]
    lax.fori_loop(0, EPW // L, acc_step, (cur0, jnp.float32(0.0)))

    lanes = jnp.arange(L, dtype=jnp.int32)

    def pack_step(jv, _):
        vec = jnp.zeros((L,), jnp.float32)
        for j in range(L):
            vec = jnp.where(lanes == j, s_loc[jv * L + j], vec)
        s_v[pl.ds(jv * L, L)] = vec
        return 0

    lax.fori_loop(0, B // L, pack_step, 0)

    pltpu.sync_copy(a_buf, ex_hbm.at[pl.ds(base, EPW)])
    pltpu.sync_copy(s_v, sump_hbm.at[w])


_sums_kernel = functools.partial(
    pl.kernel,
    out_type=[
        jax.ShapeDtypeStruct((E,), jnp.float32),
        jax.ShapeDtypeStruct((NW, B), jnp.float32),
    ],
    mesh=_mesh,
    compiler_params=pltpu.CompilerParams(needs_layout_passes=False),
    scratch_types=[
        pltpu.VMEM((EPW,), jnp.float32),
        pltpu.VMEM((EPW,), jnp.int32),
        pltpu.VMEM((NW, L), jnp.float32),
        pltpu.VMEM((B,), jnp.float32),
        pltpu.SMEM((B,), jnp.float32),
    ],
)(_sums_body)


def _norm_body(ex_hbm, ids_hbm, sump_hbm,
               out_hbm,
               ex_buf, ids_buf, sp_buf, s_buf):
    w = _wid()
    base = w * EPW
    pltpu.sync_copy(sump_hbm, sp_buf)
    pltpu.sync_copy(ex_hbm.at[pl.ds(base, EPW)], ex_buf)
    pltpu.sync_copy(ids_hbm.at[pl.ds(base, EPW)], ids_buf)

    def comb_step(j, _):
        def row_step(r, acc):
            return acc + sp_buf[r, pl.ds(j * L, L)]

        s_buf[pl.ds(j * L, L)] = lax.fori_loop(
            0, NW, row_step, jnp.zeros((L,), jnp.float32))
        return 0

    lax.fori_loop(0, B // L, comb_step, 0)

    def norm_step(i, _):
        ids = ids_buf[pl.ds(i * L, L)]
        s = plsc.load_gather(s_buf, [ids])
        ex_buf[pl.ds(i * L, L)] = ex_buf[pl.ds(i * L, L)] / s
        return 0

    lax.fori_loop(0, EPW // L, norm_step, 0)
    pltpu.sync_copy(ex_buf, out_hbm.at[pl.ds(base, EPW)])


_norm_kernel = functools.partial(
    pl.kernel,
    out_type=jax.ShapeDtypeStruct((E,), jnp.float32),
    mesh=_mesh,
    compiler_params=pltpu.CompilerParams(needs_layout_passes=False),
    scratch_types=[
        pltpu.VMEM((EPW,), jnp.float32),
        pltpu.VMEM((EPW,), jnp.int32),
        pltpu.VMEM((NW, B), jnp.float32),
        pltpu.VMEM((B,), jnp.float32),
    ],
)(_norm_body)


def kernel(x_j, x_i, edge_index, edge_index_batch, w_j, w_i, bias,
           prelu_w, lin_w, lin_b):
    src = edge_index[0].reshape(NW, NCHUNK, CHUNK)
    dst = edge_index[1].reshape(NW, NCHUNK, CHUNK)
    bias2d = bias.reshape(1, D)
    lw2d = lin_w.reshape(1, D)
    lw = lin_w.reshape(D)
    lwp = (prelu_w[0] * lin_w).reshape(D)

    xj, xib, sj, si = _project(x_j, x_i, w_j, w_i, bias2d, lw2d)
    ex, sump = _alpha_kernel(xj, xib, src, dst, lw, lwp, edge_index_batch)
    return _norm_kernel(ex, edge_index_batch, sump)
